# Initial kernel scaffold; baseline (speedup 1.0000x reference)
import functools

import jax
import jax.numpy as jnp
from jax import lax
from jax.experimental import pallas as pl
from jax.experimental.pallas import tpu as pltpu, tpu_sc as plsc

N = 10000
E = 320000
EMB = 128
LN_EPS = 1e-3

D = 144
NC = 2
NS = 16
NW = NC * NS
EP = E // NW
C = 80
J = EP // C
NP = 10240
ZR = 64
RPT = NP // NS

RB = 2000
G = N // RB


def _stage1_messages(neighbor_feats, W_msg, b_msg):
    def body(nf_ref, w_ref, b_ref, out_ref):
        r = jnp.dot(nf_ref[...], w_ref[...], preferred_element_type=jnp.float32)
        r = jnp.maximum(r + b_ref[...], 0.0)
        lane = lax.broadcasted_iota(jnp.int32, (RB, D - EMB), 1)
        pad = jnp.where(lane == 0, 1.0, 0.0).astype(jnp.float32)
        out_ref[...] = jnp.concatenate([r, pad], axis=1)

    return pl.pallas_call(
        body,
        grid=(G,),
        in_specs=[
            pl.BlockSpec((RB, EMB), lambda i: (i, 0)),
            pl.BlockSpec((EMB, EMB), lambda i: (0, 0)),
            pl.BlockSpec((1, EMB), lambda i: (0, 0)),
        ],
        out_specs=pl.BlockSpec((RB, D), lambda i: (i, 0)),
        out_shape=jax.ShapeDtypeStruct((N, D), jnp.float32),
    )(neighbor_feats, W_msg, b_msg.reshape(1, EMB))


def _stage2_scatter(msgs, src3, dst3, w3):
    mesh = plsc.VectorSubcoreMesh(core_axis_name="c", subcore_axis_name="s")

    @functools.partial(
        pl.kernel,
        mesh=mesh,
        out_type=jax.ShapeDtypeStruct((NC, NP, D), jnp.float32),
        scratch_types=[
            pltpu.VMEM((J, C), jnp.int32),
            pltpu.VMEM((J, C), jnp.int32),
            pltpu.VMEM((J, C), jnp.float32),
            pltpu.VMEM((C, D), jnp.float32),
            pltpu.VMEM((ZR, D), jnp.float32),
            pltpu.VMEM_SHARED((NP, D), jnp.float32),
            pltpu.SemaphoreType.DMA,
        ],
    )
    def k(msgs_hbm, src_hbm, dst_hbm, w_hbm, out_hbm,
          src_v, dst_v, w_v, gbuf, zbuf, acc, sem):
        cid = lax.axis_index("c")
        sid = lax.axis_index("s")
        wid = sid * NC + cid

        zeros16 = jnp.zeros((16,), jnp.float32)

        def zrow(r, carry):
            for kk in range(D // 16):
                zbuf[r, pl.ds(kk * 16, 16)] = zeros16
            return carry

        lax.fori_loop(0, ZR, zrow, 0)

        base = sid * RPT

        def zinit(i, carry):
            pltpu.sync_copy(zbuf, acc.at[pl.ds(base + i * ZR, ZR)])
            return carry

        lax.fori_loop(0, RPT // ZR, zinit, 0)

        pltpu.sync_copy(src_hbm.at[wid], src_v)
        pltpu.sync_copy(dst_hbm.at[wid], dst_v)
        pltpu.sync_copy(w_hbm.at[wid], w_v)

        plsc.subcore_barrier()

        def chunk(j, carry):
            pltpu.async_copy(msgs_hbm.at[src_v.at[j]], gbuf, sem).wait()

            def edge(e, c2):
                w = w_v[j, e]
                for kk in range(D // 16):
                    gbuf[e, pl.ds(kk * 16, 16)] = gbuf[e, pl.ds(kk * 16, 16)] * w
                return c2

            lax.fori_loop(0, C, edge, 0)
            pltpu.sync_copy(gbuf, acc.at[dst_v.at[j]], add=True)
            return carry

        lax.fori_loop(0, J, chunk, 0)

        plsc.subcore_barrier()

        pltpu.sync_copy(acc.at[pl.ds(base, RPT)],
                        out_hbm.at[cid, pl.ds(base, RPT)])

    return k(msgs, src3, dst3, w3)


def _stage3_update(partials, node_feats, W_gate, b_gate, W_upd, b_upd,
                   gamma, beta):
    def body(p0_ref, p1_ref, node_ref, wg_ref, bg_ref, wu_ref, bu_ref,
             g_ref, b_ref, out_ref):
        s = p0_ref[0] + p1_ref[0]
        agg = s[:, :EMB]
        deg = s[:, EMB:EMB + 1]
        agg = agg / jnp.maximum(deg, 1.0)
        node = node_ref[...]
        cat = jnp.concatenate([agg, node], axis=1)
        gate = jax.nn.sigmoid(
            jnp.dot(cat, wg_ref[...], preferred_element_type=jnp.float32)
            + bg_ref[...])
        upd = jnp.maximum(
            jnp.dot(cat, wu_ref[...], preferred_element_type=jnp.float32)
            + bu_ref[...], 0.0)
        o = gate * upd + (1.0 - gate) * node
        mean = jnp.mean(o, axis=1, keepdims=True)
        var = jnp.mean((o - mean) ** 2, axis=1, keepdims=True)
        out_ref[...] = ((o - mean) * lax.rsqrt(var + LN_EPS)) * g_ref[...] + b_ref[...]

    return pl.pallas_call(
        body,
        grid=(G,),
        in_specs=[
            pl.BlockSpec((1, RB, D), lambda i: (0, i, 0)),
            pl.BlockSpec((1, RB, D), lambda i: (1, i, 0)),
            pl.BlockSpec((RB, EMB), lambda i: (i, 0)),
            pl.BlockSpec((2 * EMB, EMB), lambda i: (0, 0)),
            pl.BlockSpec((1, EMB), lambda i: (0, 0)),
            pl.BlockSpec((2 * EMB, EMB), lambda i: (0, 0)),
            pl.BlockSpec((1, EMB), lambda i: (0, 0)),
            pl.BlockSpec((1, EMB), lambda i: (0, 0)),
            pl.BlockSpec((1, EMB), lambda i: (0, 0)),
        ],
        out_specs=pl.BlockSpec((RB, EMB), lambda i: (i, 0)),
        out_shape=jax.ShapeDtypeStruct((N, EMB), jnp.float32),
    )(partials, partials, node_feats, W_gate, b_gate.reshape(1, EMB),
      W_upd, b_upd.reshape(1, EMB), gamma.reshape(1, EMB),
      beta.reshape(1, EMB))


def kernel(node_feats, neighbor_feats, edge_indices, edge_weights,
           W_msg, b_msg, W_gate, b_gate, W_upd, b_upd, gamma, beta):
    msgs = _stage1_messages(neighbor_feats, W_msg, b_msg)
    src3 = edge_indices[0].astype(jnp.int32).reshape(NW, J, C)
    dst3 = edge_indices[1].astype(jnp.int32).reshape(NW, J, C)
    w3 = edge_weights.astype(jnp.float32).reshape(NW, J, C)
    partials = _stage2_scatter(msgs, src3, dst3, w3)
    return _stage3_update(partials, node_feats, W_gate, b_gate,
                          W_upd, b_upd, gamma, beta)


# speedup vs baseline: 1.0764x; 1.0764x over previous
import jax
import jax.numpy as jnp
from jax import lax
from jax.experimental import pallas as pl

N = 10000
E = 320000
EMB = 128
LN_EPS = 1e-3

NP = 10240
RB = 2000
G = N // RB
RB3 = 1024
G3 = NP // RB3


def _stage1_messages(neighbor_feats, W_msg, b_msg):
    def body(nf_ref, w_ref, b_ref, out_ref):
        r = jnp.dot(nf_ref[...], w_ref[...], preferred_element_type=jnp.float32)
        out_ref[...] = jnp.maximum(r + b_ref[...], 0.0)

    return pl.pallas_call(
        body,
        grid=(G,),
        in_specs=[
            pl.BlockSpec((RB, EMB), lambda i: (i, 0)),
            pl.BlockSpec((EMB, EMB), lambda i: (0, 0)),
            pl.BlockSpec((1, EMB), lambda i: (0, 0)),
        ],
        out_specs=pl.BlockSpec((RB, EMB), lambda i: (i, 0)),
        out_shape=jax.ShapeDtypeStruct((N, EMB), jnp.float32),
    )(neighbor_feats, W_msg, b_msg.reshape(1, EMB))


def _stage3_update(agg, deg, node_feats, W_gate, b_gate, W_upd, b_upd,
                   gamma, beta):
    def body(agg_ref, deg_ref, node_ref, wg_ref, bg_ref, wu_ref, bu_ref,
             g_ref, b_ref, out_ref):
        agg_n = agg_ref[...] / jnp.maximum(deg_ref[...], 1.0)
        node = node_ref[...]
        cat = jnp.concatenate([agg_n, node], axis=1)
        gate = jax.nn.sigmoid(
            jnp.dot(cat, wg_ref[...], preferred_element_type=jnp.float32)
            + bg_ref[...])
        upd = jnp.maximum(
            jnp.dot(cat, wu_ref[...], preferred_element_type=jnp.float32)
            + bu_ref[...], 0.0)
        o = gate * upd + (1.0 - gate) * node
        mean = jnp.mean(o, axis=1, keepdims=True)
        var = jnp.mean((o - mean) ** 2, axis=1, keepdims=True)
        out_ref[...] = ((o - mean) * lax.rsqrt(var + LN_EPS)) * g_ref[...] + b_ref[...]

    return pl.pallas_call(
        body,
        grid=(G3,),
        in_specs=[
            pl.BlockSpec((RB3, EMB), lambda i: (i, 0)),
            pl.BlockSpec((RB3, 1), lambda i: (i, 0)),
            pl.BlockSpec((RB3, EMB), lambda i: (i, 0)),
            pl.BlockSpec((2 * EMB, EMB), lambda i: (0, 0)),
            pl.BlockSpec((1, EMB), lambda i: (0, 0)),
            pl.BlockSpec((2 * EMB, EMB), lambda i: (0, 0)),
            pl.BlockSpec((1, EMB), lambda i: (0, 0)),
            pl.BlockSpec((1, EMB), lambda i: (0, 0)),
            pl.BlockSpec((1, EMB), lambda i: (0, 0)),
        ],
        out_specs=pl.BlockSpec((RB3, EMB), lambda i: (i, 0)),
        out_shape=jax.ShapeDtypeStruct((NP, EMB), jnp.float32),
    )(agg, deg, node_feats, W_gate, b_gate.reshape(1, EMB),
      W_upd, b_upd.reshape(1, EMB), gamma.reshape(1, EMB),
      beta.reshape(1, EMB))


def kernel(node_feats, neighbor_feats, edge_indices, edge_weights,
           W_msg, b_msg, W_gate, b_gate, W_upd, b_upd, gamma, beta):
    msgs = _stage1_messages(neighbor_feats, W_msg, b_msg)
    src = edge_indices[0].astype(jnp.int32)
    dst = edge_indices[1].astype(jnp.int32)
    w = edge_weights.astype(jnp.float32)
    wm = jnp.take(msgs, src, axis=0) * w
    agg = jnp.zeros((N, EMB), jnp.float32).at[dst].add(wm)
    deg = jnp.zeros((N, 1), jnp.float32).at[dst].add(w)
    agg_p = jnp.pad(agg, ((0, NP - N), (0, 0)))
    deg_p = jnp.pad(deg, ((0, NP - N), (0, 0)))
    node_p = jnp.pad(node_feats, ((0, NP - N), (0, 0)))
    out = _stage3_update(agg_p, deg_p, node_p, W_gate, b_gate,
                         W_upd, b_upd, gamma, beta)
    return out[:N]
